# baseline (speedup 1.0000x reference)
import functools

import jax
import jax.numpy as jnp
from jax import lax
from jax.experimental import pallas as pl
from jax.experimental.pallas import tpu as pltpu
from jax.experimental.pallas import tpu_sc as plsc

N = 10000
E = 160000
D = 256
H = 256
DE = 128
G = 8
DU = 128

NC = 2
NS = 16
NW = NC * NS
L = 16

N_PAD = 10240
E_PAD = 163840
EW = E // NW
EWP = E_PAD // NW
ET = E_PAD // NS


def _mesh():
    return plsc.VectorSubcoreMesh(core_axis_name="c", subcore_axis_name="s")


def _wid():
    return lax.axis_index("s") * NC + lax.axis_index("c")


def _tc_pre_body(x_ref, xsrc_ref, b_ref, wrc_ref, wu_ref, u_ref, be1_ref,
                 prow_ref, pcol_ref, xn_ref, xsn_ref):
    x = x_ref[...]
    xs = xsrc_ref[...]
    b = b_ref[0, 0, :]
    mask = (b[:, None] == lax.broadcasted_iota(jnp.int32, (x.shape[0], G), 1)
            ).astype(jnp.float32)
    u8 = jnp.dot(u_ref[...], wu_ref[...], preferred_element_type=jnp.float32)
    mm = jnp.dot(x, wrc_ref[...], preferred_element_type=jnp.float32)
    prow_ref[...] = mm[:, :H] + jnp.dot(mask, u8) + be1_ref[...]
    pcol_ref[...] = mm[:, H:]
    nx = jnp.sqrt(jnp.sum(x * x, axis=1, keepdims=True))
    xn_ref[...] = x / jnp.maximum(nx, 1e-20)
    ns = jnp.sqrt(jnp.sum(xs * xs, axis=1, keepdims=True))
    xsn_ref[...] = xs / jnp.maximum(ns, 1e-20)


def _tc_pre(x, x_src, batch3, W_rc, W_u, u, be1):
    nb = 10
    blk = N // nb
    return pl.pallas_call(
        _tc_pre_body,
        grid=(nb,),
        in_specs=[
            pl.BlockSpec((blk, D), lambda i: (i, 0)),
            pl.BlockSpec((blk, D), lambda i: (i, 0)),
            pl.BlockSpec((1, 1, blk), lambda i: (i, 0, 0)),
            pl.BlockSpec((D, 2 * H), lambda i: (0, 0)),
            pl.BlockSpec((DU, H), lambda i: (0, 0)),
            pl.BlockSpec((G, DU), lambda i: (0, 0)),
            pl.BlockSpec((1, H), lambda i: (0, 0)),
        ],
        out_specs=[
            pl.BlockSpec((blk, H), lambda i: (i, 0)),
            pl.BlockSpec((blk, H), lambda i: (i, 0)),
            pl.BlockSpec((blk, D), lambda i: (i, 0)),
            pl.BlockSpec((blk, D), lambda i: (i, 0)),
        ],
        out_shape=[
            jax.ShapeDtypeStruct((N, H), jnp.float32),
            jax.ShapeDtypeStruct((N, H), jnp.float32),
            jax.ShapeDtypeStruct((N, D), jnp.float32),
            jax.ShapeDtypeStruct((N, D), jnp.float32),
        ],
    )(x, x_src, batch3, W_rc, W_u, u, be1)


def _sc_edge_combine(P_row, P_col, row, col):
    C = 200
    K = EW // C

    @functools.partial(
        pl.kernel,
        out_type=jax.ShapeDtypeStruct((E, H), jnp.float32),
        mesh=_mesh(),
        compiler_params=pltpu.CompilerParams(needs_layout_passes=False),
        scratch_types=[
            pltpu.VMEM((C,), jnp.int32),
            pltpu.VMEM((C,), jnp.int32),
            pltpu.VMEM((C, H), jnp.float32),
            pltpu.VMEM((C, H), jnp.float32),
            pltpu.SemaphoreType.DMA,
            pltpu.SemaphoreType.DMA,
        ],
    )
    def k(prow_hbm, pcol_hbm, row_hbm, col_hbm, out_hbm,
          ia_v, ib_v, ba_v, bb_v, sem1, sem2):
        w = _wid()

        def chunk(kk, _):
            base = w * EW + kk * C
            pltpu.sync_copy(row_hbm.at[pl.ds(base, C)], ia_v)
            pltpu.sync_copy(col_hbm.at[pl.ds(base, C)], ib_v)
            cp1 = pltpu.async_copy(prow_hbm.at[ia_v], ba_v, sem1)
            cp2 = pltpu.async_copy(pcol_hbm.at[ib_v], bb_v, sem2)
            cp1.wait()
            cp2.wait()

            def addrow(r, _):
                for c in range(H // L):
                    ba_v[r, pl.ds(c * L, L)] = (
                        ba_v[r, pl.ds(c * L, L)] + bb_v[r, pl.ds(c * L, L)])
                return 0

            lax.fori_loop(0, C, addrow, 0)
            pltpu.sync_copy(ba_v, out_hbm.at[pl.ds(base, C)])
            return 0

        lax.fori_loop(0, K, chunk, 0)

    return k(P_row, P_col, row, col)


def _tc_edge_mlp_body(ge_ref, ea_ref, wa_ref, we2_ref, be2_ref, out_ref):
    h = jax.nn.relu(
        ge_ref[...]
        + jnp.dot(ea_ref[...], wa_ref[...], preferred_element_type=jnp.float32))
    out_ref[...] = (
        jnp.dot(h, we2_ref[...], preferred_element_type=jnp.float32)
        + be2_ref[...])


def _tc_edge_mlp(G_e, edge_attr, W_attr, We2, be2):
    nb = 160
    blk = E // nb
    return pl.pallas_call(
        _tc_edge_mlp_body,
        grid=(nb,),
        in_specs=[
            pl.BlockSpec((blk, H), lambda i: (i, 0)),
            pl.BlockSpec((blk, 16), lambda i: (i, 0)),
            pl.BlockSpec((16, H), lambda i: (0, 0)),
            pl.BlockSpec((H, DE), lambda i: (0, 0)),
            pl.BlockSpec((1, DE), lambda i: (0, 0)),
        ],
        out_specs=pl.BlockSpec((blk, DE), lambda i: (i, 0)),
        out_shape=jax.ShapeDtypeStruct((E, DE), jnp.float32),
    )(G_e, edge_attr, W_attr, We2, be2)


def _sc_cos_denom(xn, xsn, dst_p, src_p):
    C = 160
    K = EWP // C
    GR = C // L
    STRIPE = N_PAD // NS

    @functools.partial(
        pl.kernel,
        out_type=[
            jax.ShapeDtypeStruct((E_PAD,), jnp.float32),
            jax.ShapeDtypeStruct((NC, NS, STRIPE), jnp.float32),
        ],
        mesh=_mesh(),
        compiler_params=pltpu.CompilerParams(needs_layout_passes=False),
        scratch_types=[
            pltpu.VMEM((C,), jnp.int32),
            pltpu.VMEM((C,), jnp.int32),
            pltpu.VMEM((C, D), jnp.float32),
            pltpu.VMEM((C, D), jnp.float32),
            pltpu.VMEM((C,), jnp.float32),
            pltpu.VMEM((STRIPE,), jnp.float32),
            pltpu.VMEM_SHARED((N_PAD,), jnp.float32),
            pltpu.SemaphoreType.DMA,
            pltpu.SemaphoreType.DMA,
        ],
    )
    def k(xn_hbm, xsn_hbm, dst_hbm, src_hbm, ex_hbm, den_hbm,
          id_v, is_v, bd_v, bs_v, exc_v, z_v, den_sh, sem1, sem2):
        cid = lax.axis_index("c")
        sid = lax.axis_index("s")
        w = sid * NC + cid

        def zloop(i, _):
            z_v[pl.ds(i * L, L)] = jnp.zeros((L,), jnp.float32)
            return 0

        lax.fori_loop(0, STRIPE // L, zloop, 0)
        pltpu.sync_copy(z_v, den_sh.at[pl.ds(sid * STRIPE, STRIPE)])
        plsc.subcore_barrier()

        def chunk(kk, _):
            base = w * EWP + kk * C
            pltpu.sync_copy(dst_hbm.at[pl.ds(base, C)], id_v)
            pltpu.sync_copy(src_hbm.at[pl.ds(base, C)], is_v)
            cp1 = pltpu.async_copy(xn_hbm.at[id_v], bd_v, sem1)
            cp2 = pltpu.async_copy(xsn_hbm.at[is_v], bs_v, sem2)
            cp1.wait()
            cp2.wait()

            def group(g, _):
                rows = g * L + lax.iota(jnp.int32, L)

                cos = jnp.zeros((L,), jnp.float32)
                for c in range(D):
                    cc = jnp.full((L,), c, jnp.int32)
                    va = plsc.load_gather(bd_v, [rows, cc])
                    vb = plsc.load_gather(bs_v, [rows, cc])
                    cos = cos + va * vb
                eid = base + rows
                ex = jnp.where(eid < E, jnp.exp(cos - 1.0), 0.0)
                exc_v[pl.ds(g * L, L)] = ex
                return 0

            lax.fori_loop(0, GR, group, 0)
            pltpu.sync_copy(exc_v, den_sh.at[id_v], add=True)
            pltpu.sync_copy(exc_v, ex_hbm.at[pl.ds(base, C)])
            return 0

        lax.fori_loop(0, K, chunk, 0)
        plsc.subcore_barrier()
        pltpu.sync_copy(den_sh.at[pl.ds(sid * STRIPE, STRIPE)],
                        den_hbm.at[cid, sid])

    return k(xn, xsn, dst_p, src_p)


def _tc_denom_merge_body(d_ref, out_ref):
    out_ref[...] = d_ref[0] + d_ref[1] + 1e-16


def _tc_denom_merge(den_part):
    d3 = den_part.reshape(NC, 8, N_PAD // 8)
    return pl.pallas_call(
        _tc_denom_merge_body,
        grid=(1,),
        in_specs=[pl.BlockSpec((NC, 8, N_PAD // 8), lambda i: (0, 0, 0))],
        out_specs=pl.BlockSpec((8, N_PAD // 8), lambda i: (0, 0)),
        out_shape=jax.ShapeDtypeStruct((8, N_PAD // 8), jnp.float32),
    )(d3)


def _sc_attn_scatter(ex, den, dst_p, src2, xs_cat):
    C = 128
    K = ET // C
    GR = C // L
    HDE = DE
    SROWS = N_PAD // NS

    @functools.partial(
        pl.kernel,
        out_type=jax.ShapeDtypeStruct((NC, N_PAD, HDE), jnp.float32),
        mesh=_mesh(),
        compiler_params=pltpu.CompilerParams(needs_layout_passes=False),
        scratch_types=[
            pltpu.VMEM((C,), jnp.int32),
            pltpu.VMEM((C,), jnp.int32),
            pltpu.VMEM((C,), jnp.float32),
            pltpu.VMEM((C, HDE), jnp.float32),
            pltpu.VMEM((N_PAD,), jnp.float32),
            pltpu.VMEM_SHARED((N_PAD, HDE), jnp.float32),
            pltpu.SemaphoreType.DMA,
        ],
    )
    def k(ex_hbm, den_hbm, dst_hbm, src2_hbm, xs_hbm, out_hbm,
          id_v, is_v, ex_v, rows_v, den_v, acc_sh, sem1):
        cid = lax.axis_index("c")
        sid = lax.axis_index("s")

        pltpu.sync_copy(den_hbm.at[pl.ds(0, N_PAD)], den_v)

        def zrow(r, _):
            for c in range(HDE // L):
                rows_v[r, pl.ds(c * L, L)] = jnp.zeros((L,), jnp.float32)
            return 0

        lax.fori_loop(0, C, zrow, 0)
        for p in range(SROWS // C):
            pltpu.sync_copy(rows_v, acc_sh.at[pl.ds(sid * SROWS + p * C, C)])
        plsc.subcore_barrier()

        def chunk(kk, _):
            base = sid * ET + kk * C
            pltpu.sync_copy(dst_hbm.at[pl.ds(base, C)], id_v)
            pltpu.sync_copy(src2_hbm.at[cid, pl.ds(base, C)], is_v)
            pltpu.sync_copy(ex_hbm.at[pl.ds(base, C)], ex_v)
            pltpu.async_copy(xs_hbm.at[is_v], rows_v, sem1).wait()

            def group(g, _):
                rows = g * L + lax.iota(jnp.int32, L)
                dd = id_v[pl.ds(g * L, L)]
                dn = plsc.load_gather(den_v, [dd])
                attn = ex_v[pl.ds(g * L, L)] / dn

                for c in range(HDE):
                    cc = jnp.full((L,), c, jnp.int32)
                    v = plsc.load_gather(rows_v, [rows, cc])
                    plsc.store_scatter(rows_v, [rows, cc], v * attn)
                return 0

            lax.fori_loop(0, GR, group, 0)
            pltpu.sync_copy(rows_v, acc_sh.at[id_v], add=True)
            return 0

        lax.fori_loop(0, K, chunk, 0)
        plsc.subcore_barrier()
        for p in range(SROWS // C):
            r0 = sid * SROWS + p * C
            pltpu.sync_copy(acc_sh.at[pl.ds(r0, C)],
                            out_hbm.at[cid, pl.ds(r0, C)])

    return k(ex, den, dst_p, src2, xs_cat)


def _sc_agg_scatter(edge_attr_new, col):
    C = 200
    ECORE = E // NC
    ETILE = ECORE // NS
    K = ETILE // C
    ZC = 128
    SROWS = N_PAD // NS

    @functools.partial(
        pl.kernel,
        out_type=jax.ShapeDtypeStruct((NC, N_PAD, DE), jnp.float32),
        mesh=_mesh(),
        compiler_params=pltpu.CompilerParams(needs_layout_passes=False),
        scratch_types=[
            pltpu.VMEM((C,), jnp.int32),
            pltpu.VMEM((C, DE), jnp.float32),
            pltpu.VMEM_SHARED((N_PAD, DE), jnp.float32),
        ],
    )
    def k(ea_hbm, col_hbm, out_hbm, ic_v, rows_v, acc_sh):
        cid = lax.axis_index("c")
        sid = lax.axis_index("s")

        def zrow(r, _):
            for c in range(DE // L):
                rows_v[r, pl.ds(c * L, L)] = jnp.zeros((L,), jnp.float32)
            return 0

        lax.fori_loop(0, ZC, zrow, 0)
        for p in range(SROWS // ZC):
            pltpu.sync_copy(rows_v.at[pl.ds(0, ZC)],
                            acc_sh.at[pl.ds(sid * SROWS + p * ZC, ZC)])
        plsc.subcore_barrier()

        def chunk(kk, _):
            base = cid * ECORE + sid * ETILE + kk * C
            pltpu.sync_copy(col_hbm.at[pl.ds(base, C)], ic_v)
            pltpu.sync_copy(ea_hbm.at[pl.ds(base, C)], rows_v)
            pltpu.sync_copy(rows_v, acc_sh.at[ic_v], add=True)
            return 0

        lax.fori_loop(0, K, chunk, 0)
        plsc.subcore_barrier()
        for p in range(SROWS // ZC):
            r0 = sid * SROWS + p * ZC
            pltpu.sync_copy(acc_sh.at[pl.ds(r0, ZC)],
                            out_hbm.at[cid, pl.ds(r0, ZC)])

    return k(edge_attr_new, col)


def _tc_node_body(x_ref, alo_ref, ahi_ref, p0_ref, p1_ref, b_ref,
                  ax_ref, aal_ref, aah_ref, aagg_ref, au_ref, u_ref, bn1_ref,
                  wn2_ref, bn2_ref, wg1u_ref, wg1m_ref, bg1_ref,
                  wg2_ref, bg2_ref, xnew_ref, unew_ref, sum_ref, cnt_ref):
    i = pl.program_id(0)
    x = x_ref[...]
    alo = alo_ref[0]
    ahi = ahi_ref[0]
    p0 = p0_ref[0]
    p1 = p1_ref[0]
    b = b_ref[0, 0, :]
    mask = (b[:, None] == lax.broadcasted_iota(jnp.int32, (x.shape[0], G), 1)
            ).astype(jnp.float32)
    ua8 = jnp.dot(u_ref[...], au_ref[...], preferred_element_type=jnp.float32)
    pre = (jnp.dot(x, ax_ref[...], preferred_element_type=jnp.float32)
           + jnp.dot(alo, aal_ref[...],
                     preferred_element_type=jnp.float32)
           + jnp.dot(ahi, aah_ref[...],
                     preferred_element_type=jnp.float32)
           + jnp.dot(p0 + p1, aagg_ref[...],
                     preferred_element_type=jnp.float32)
           + jnp.dot(mask, ua8) + bn1_ref[...])
    n_h = jax.nn.relu(pre)
    x_new = (jnp.dot(n_h, wn2_ref[...], preferred_element_type=jnp.float32)
             + bn2_ref[...])
    xnew_ref[...] = x_new

    @pl.when(i == 0)
    def _():
        sum_ref[...] = jnp.zeros_like(sum_ref)
        cnt_ref[...] = jnp.zeros_like(cnt_ref)

    sum_ref[...] += lax.dot_general(
        mask, x_new, (((0,), (0,)), ((), ())),
        preferred_element_type=jnp.float32)
    cnt_ref[...] += jnp.broadcast_to(
        jnp.sum(mask, axis=0)[:, None], cnt_ref.shape)

    @pl.when(i == pl.num_programs(0) - 1)
    def _():
        counts = jnp.clip(cnt_ref[:, :1], 1.0, None)
        mean_x = sum_ref[...] / counts
        g_h = jax.nn.relu(
            jnp.dot(u_ref[...], wg1u_ref[...],
                    preferred_element_type=jnp.float32)
            + jnp.dot(mean_x, wg1m_ref[...],
                      preferred_element_type=jnp.float32)
            + bg1_ref[...])
        unew_ref[...] = (
            jnp.dot(g_h, wg2_ref[...], preferred_element_type=jnp.float32)
            + bg2_ref[...])


def _tc_node_global(x, a_halves, agg_part, batch3, A_x, A_a_lo, A_a_hi,
                    A_agg, A_u, u, bn1, Wn2, bn2, Wg1u, Wg1m, bg1, Wg2, bg2):
    nb = 10
    blk = N // nb
    full = lambda shape: pl.BlockSpec(shape, lambda i: tuple(0 for _ in shape))
    return pl.pallas_call(
        _tc_node_body,
        grid=(nb,),
        in_specs=[
            pl.BlockSpec((blk, D), lambda i: (i, 0)),
            pl.BlockSpec((1, blk, DE), lambda i: (0, i, 0)),
            pl.BlockSpec((1, blk, DE), lambda i: (1, i, 0)),
            pl.BlockSpec((1, blk, DE), lambda i: (0, i, 0)),
            pl.BlockSpec((1, blk, DE), lambda i: (1, i, 0)),
            pl.BlockSpec((1, 1, blk), lambda i: (i, 0, 0)),
            full((D, H)), full((DE, H)), full((DE, H)), full((DE, H)),
            full((DU, H)), full((G, DU)), full((1, H)),
            full((H, D)), full((1, D)),
            full((DU, H)), full((D, H)), full((1, H)),
            full((H, DU)), full((1, DU)),
        ],
        out_specs=[
            pl.BlockSpec((blk, D), lambda i: (i, 0)),
            pl.BlockSpec((G, DU), lambda i: (0, 0)),
        ],
        out_shape=[
            jax.ShapeDtypeStruct((N, D), jnp.float32),
            jax.ShapeDtypeStruct((G, DU), jnp.float32),
        ],
        scratch_shapes=[
            pltpu.VMEM((G, D), jnp.float32),
            pltpu.VMEM((G, DE), jnp.float32),
        ],
    )(x, a_halves, a_halves, agg_part, agg_part, batch3, A_x, A_a_lo,
      A_a_hi, A_agg, A_u, u, bn1, Wn2, bn2, Wg1u, Wg1m, bg1, Wg2, bg2)


def kernel(x, x_src, edge_index, cg_edge_index, edge_attr, u, batch,
           batch_src, We1, be1, We2, be2, Wn1, bn1, Wn2, bn2,
           Wg1, bg1, Wg2, bg2):
    row = edge_index[0]
    col = edge_index[1]
    src = cg_edge_index[0]
    dst = cg_edge_index[1]
    src_p = jnp.pad(src, (0, E_PAD - E))
    dst_p = jnp.pad(dst, (0, E_PAD - E))
    batch3 = batch.reshape(10, 1, N // 10)

    W_rc = jnp.concatenate([We1[:D], We1[D:2 * D]], axis=1)
    W_attr = We1[2 * D:2 * D + 16]
    W_u = We1[2 * D + 16:]
    A_x = Wn1[:D]
    A_a_lo = Wn1[D:D + DE]
    A_a_hi = Wn1[D + DE:2 * D]
    A_agg = Wn1[2 * D:2 * D + DE]
    A_u = Wn1[2 * D + DE:]
    Wg1u = Wg1[:DU]
    Wg1m = Wg1[DU:]

    P_row, P_col, xn, xsn = _tc_pre(
        x, x_src, batch3, W_rc, W_u, u, be1.reshape(1, H))

    G_e = _sc_edge_combine(P_row, P_col, row, col)
    edge_attr_new = _tc_edge_mlp(
        G_e, edge_attr, W_attr, We2, be2.reshape(1, DE))

    ex, den_part = _sc_cos_denom(xn, xsn, dst_p, src_p)
    den = _tc_denom_merge(den_part).reshape(N_PAD)

    xs_cat = jnp.concatenate([x_src[:, :DE], x_src[:, DE:]], axis=0)
    src2 = jnp.stack([src_p, src_p + N], axis=0)
    a_halves = _sc_attn_scatter(ex, den, dst_p, src2, xs_cat)

    agg_part = _sc_agg_scatter(edge_attr_new, col)

    x_new, u_new = _tc_node_global(
        x, a_halves, agg_part, batch3,
        A_x, A_a_lo, A_a_hi, A_agg, A_u, u, bn1.reshape(1, H), Wn2,
        bn2.reshape(1, D), Wg1u, Wg1m, bg1.reshape(1, H), Wg2,
        bg2.reshape(1, DU))

    return (x_new, edge_attr_new, u_new)


# speedup vs baseline: 3.9671x; 3.9671x over previous
import functools

import jax
import jax.numpy as jnp
from jax import lax
from jax.experimental import pallas as pl
from jax.experimental.pallas import tpu as pltpu
from jax.experimental.pallas import tpu_sc as plsc

N = 10000
E = 160000
D = 256
H = 256
DE = 128
G = 8
DU = 128

NC = 2
NS = 16
NW = NC * NS
L = 16

N_PAD = 10240
E_PAD = 163840
EW = E // NW
EWP = E_PAD // NW
ET = E_PAD // NS


def _mesh():
    return plsc.VectorSubcoreMesh(core_axis_name="c", subcore_axis_name="s")


def _wid():
    return lax.axis_index("s") * NC + lax.axis_index("c")


def _tc_pre_body(x_ref, b_ref, wrc_ref, wu_ref, u_ref, be1_ref,
                 prow_ref, pcol_ref, xn_ref):
    x = x_ref[...]
    b = b_ref[0, 0, :]
    mask = (b[:, None] == lax.broadcasted_iota(jnp.int32, (x.shape[0], G), 1)
            ).astype(jnp.float32)
    u8 = jnp.dot(u_ref[...], wu_ref[...], preferred_element_type=jnp.float32)
    mm = jnp.dot(x, wrc_ref[...], preferred_element_type=jnp.float32)
    prow_ref[...] = mm[:, :H] + jnp.dot(mask, u8) + be1_ref[...]
    pcol_ref[...] = mm[:, H:]
    nx = jnp.sqrt(jnp.sum(x * x, axis=1, keepdims=True))
    xn_ref[...] = x / jnp.maximum(nx, 1e-20)


def _tc_pre(x, batch3, W_rc, W_u, u, be1):
    nb = 10
    blk = N // nb
    return pl.pallas_call(
        _tc_pre_body,
        grid=(nb,),
        in_specs=[
            pl.BlockSpec((blk, D), lambda i: (i, 0)),
            pl.BlockSpec((1, 1, blk), lambda i: (i, 0, 0)),
            pl.BlockSpec((D, 2 * H), lambda i: (0, 0)),
            pl.BlockSpec((DU, H), lambda i: (0, 0)),
            pl.BlockSpec((G, DU), lambda i: (0, 0)),
            pl.BlockSpec((1, H), lambda i: (0, 0)),
        ],
        out_specs=[
            pl.BlockSpec((blk, H), lambda i: (i, 0)),
            pl.BlockSpec((blk, H), lambda i: (i, 0)),
            pl.BlockSpec((blk, D), lambda i: (i, 0)),
        ],
        out_shape=[
            jax.ShapeDtypeStruct((N, H), jnp.float32),
            jax.ShapeDtypeStruct((N, H), jnp.float32),
            jax.ShapeDtypeStruct((N, D), jnp.float32),
        ],
    )(x, batch3, W_rc, W_u, u, be1)


def _sc_edge_combine(P_row, P_col, row, col):
    C = 200
    K = EW // C

    @functools.partial(
        pl.kernel,
        out_type=jax.ShapeDtypeStruct((E, H), jnp.float32),
        mesh=_mesh(),
        compiler_params=pltpu.CompilerParams(needs_layout_passes=False),
        scratch_types=[
            pltpu.VMEM((C,), jnp.int32),
            pltpu.VMEM((C,), jnp.int32),
            pltpu.VMEM((C, H), jnp.float32),
            pltpu.VMEM((C, H), jnp.float32),
            pltpu.SemaphoreType.DMA,
            pltpu.SemaphoreType.DMA,
        ],
    )
    def k(prow_hbm, pcol_hbm, row_hbm, col_hbm, out_hbm,
          ia_v, ib_v, ba_v, bb_v, sem1, sem2):
        w = _wid()

        def chunk(kk, _):
            base = w * EW + kk * C
            pltpu.sync_copy(row_hbm.at[pl.ds(base, C)], ia_v)
            pltpu.sync_copy(col_hbm.at[pl.ds(base, C)], ib_v)
            cp1 = pltpu.async_copy(prow_hbm.at[ia_v], ba_v, sem1)
            cp2 = pltpu.async_copy(pcol_hbm.at[ib_v], bb_v, sem2)
            cp1.wait()
            cp2.wait()

            def addrow(r, _):
                for c in range(H // L):
                    ba_v[r, pl.ds(c * L, L)] = (
                        ba_v[r, pl.ds(c * L, L)] + bb_v[r, pl.ds(c * L, L)])
                return 0

            lax.fori_loop(0, C, addrow, 0)
            pltpu.sync_copy(ba_v, out_hbm.at[pl.ds(base, C)])
            return 0

        lax.fori_loop(0, K, chunk, 0)

    return k(P_row, P_col, row, col)


def _tc_edge_mlp_body(ge_ref, ea_ref, wa_ref, we2_ref, be2_ref, out_ref):
    h = jax.nn.relu(
        ge_ref[...]
        + jnp.dot(ea_ref[...], wa_ref[...], preferred_element_type=jnp.float32))
    out_ref[...] = (
        jnp.dot(h, we2_ref[...], preferred_element_type=jnp.float32)
        + be2_ref[...])


def _tc_edge_mlp(G_e, edge_attr, W_attr, We2, be2):
    nb = 160
    blk = E // nb
    return pl.pallas_call(
        _tc_edge_mlp_body,
        grid=(nb,),
        in_specs=[
            pl.BlockSpec((blk, H), lambda i: (i, 0)),
            pl.BlockSpec((blk, 16), lambda i: (i, 0)),
            pl.BlockSpec((16, H), lambda i: (0, 0)),
            pl.BlockSpec((H, DE), lambda i: (0, 0)),
            pl.BlockSpec((1, DE), lambda i: (0, 0)),
        ],
        out_specs=pl.BlockSpec((blk, DE), lambda i: (i, 0)),
        out_shape=jax.ShapeDtypeStruct((E, DE), jnp.float32),
    )(G_e, edge_attr, W_attr, We2, be2)


def _sc_gather_pairs(xn, xsrc, dst, src):
    C = 200
    K = EW // C

    @functools.partial(
        pl.kernel,
        out_type=[
            jax.ShapeDtypeStruct((E, D), jnp.float32),
            jax.ShapeDtypeStruct((E, D), jnp.float32),
        ],
        mesh=_mesh(),
        compiler_params=pltpu.CompilerParams(needs_layout_passes=False),
        scratch_types=[
            pltpu.VMEM((C,), jnp.int32),
            pltpu.VMEM((C,), jnp.int32),
            pltpu.VMEM((C, D), jnp.float32),
            pltpu.VMEM((C, D), jnp.float32),
            pltpu.SemaphoreType.DMA,
            pltpu.SemaphoreType.DMA,
        ],
    )
    def k(xn_hbm, xsrc_hbm, dst_hbm, src_hbm,
          xd_hbm, xs_hbm, id_v, is_v, bd_v, bs_v, sem1, sem2):
        w = _wid()

        def chunk(kk, _):
            base = w * EW + kk * C
            pltpu.sync_copy(dst_hbm.at[pl.ds(base, C)], id_v)
            pltpu.sync_copy(src_hbm.at[pl.ds(base, C)], is_v)
            cp1 = pltpu.async_copy(xn_hbm.at[id_v], bd_v, sem1)
            cp2 = pltpu.async_copy(xsrc_hbm.at[is_v], bs_v, sem2)
            cp1.wait()
            cp2.wait()
            pltpu.sync_copy(bd_v, xd_hbm.at[pl.ds(base, C)])
            pltpu.sync_copy(bs_v, xs_hbm.at[pl.ds(base, C)])
            return 0

        lax.fori_loop(0, K, chunk, 0)

    return k(xn, xsrc, dst, src)


def _tc_cos_ex_body(xd_ref, xs_ref, ex_ref, s_ref):
    xs = xs_ref[...]
    ns = jnp.sqrt(jnp.sum(xs * xs, axis=1))
    c = jnp.sum(xd_ref[...] * xs, axis=1) / jnp.maximum(ns, 1e-20)
    ex = jnp.exp(c - 1.0)
    ex_ref[0, 0, :] = ex
    s_ref[0] = ex[:, None] * xs[:, :DE]
    s_ref[1] = ex[:, None] * xs[:, DE:]


def _tc_cos_ex(XD, XS):
    nb = 160
    blk = E // nb
    return pl.pallas_call(
        _tc_cos_ex_body,
        grid=(nb,),
        in_specs=[
            pl.BlockSpec((blk, D), lambda i: (i, 0)),
            pl.BlockSpec((blk, D), lambda i: (i, 0)),
        ],
        out_specs=[
            pl.BlockSpec((1, 1, blk), lambda i: (i, 0, 0)),
            pl.BlockSpec((NC, blk, DE), lambda i: (0, i, 0)),
        ],
        out_shape=[
            jax.ShapeDtypeStruct((nb, 1, blk), jnp.float32),
            jax.ShapeDtypeStruct((NC, E, DE), jnp.float32),
        ],
    )(XD, XS)


def _sc_den_scatter(ex, dst):
    C = 200
    ECORE = E // NC
    ETILE = ECORE // NS
    K = ETILE // C
    STRIPE = N_PAD // NS

    @functools.partial(
        pl.kernel,
        out_type=jax.ShapeDtypeStruct((NC, NS, STRIPE), jnp.float32),
        mesh=_mesh(),
        compiler_params=pltpu.CompilerParams(needs_layout_passes=False),
        scratch_types=[
            pltpu.VMEM((C,), jnp.int32),
            pltpu.VMEM((C,), jnp.float32),
            pltpu.VMEM((STRIPE,), jnp.float32),
            pltpu.VMEM_SHARED((N_PAD,), jnp.float32),
        ],
    )
    def k(ex_hbm, dst_hbm, den_hbm, id_v, ev_v, z_v, den_sh):
        cid = lax.axis_index("c")
        sid = lax.axis_index("s")

        def zloop(i, _):
            z_v[pl.ds(i * L, L)] = jnp.zeros((L,), jnp.float32)
            return 0

        lax.fori_loop(0, STRIPE // L, zloop, 0)
        pltpu.sync_copy(z_v, den_sh.at[pl.ds(sid * STRIPE, STRIPE)])
        plsc.subcore_barrier()

        def chunk(kk, _):
            base = cid * ECORE + sid * ETILE + kk * C
            pltpu.sync_copy(dst_hbm.at[pl.ds(base, C)], id_v)
            pltpu.sync_copy(ex_hbm.at[pl.ds(base, C)], ev_v)
            pltpu.sync_copy(ev_v, den_sh.at[id_v], add=True)
            return 0

        lax.fori_loop(0, K, chunk, 0)
        plsc.subcore_barrier()
        pltpu.sync_copy(den_sh.at[pl.ds(sid * STRIPE, STRIPE)],
                        den_hbm.at[cid, sid])

    return k(ex, dst)


def _tc_denom_merge_body(d_ref, out_ref):
    out_ref[...] = d_ref[0] + d_ref[1] + 1e-16


def _tc_denom_merge(den_part):
    d3 = den_part.reshape(NC, 8, N_PAD // 8)
    return pl.pallas_call(
        _tc_denom_merge_body,
        grid=(1,),
        in_specs=[pl.BlockSpec((NC, 8, N_PAD // 8), lambda i: (0, 0, 0))],
        out_specs=pl.BlockSpec((8, N_PAD // 8), lambda i: (0, 0)),
        out_shape=jax.ShapeDtypeStruct((8, N_PAD // 8), jnp.float32),
    )(d3)


def _sc_a_scatter(S_stk, dst):
    C = 200
    ETILE = E // NS
    K = ETILE // C
    ZC = 128
    SROWS = N_PAD // NS

    @functools.partial(
        pl.kernel,
        out_type=jax.ShapeDtypeStruct((NC, N_PAD, DE), jnp.float32),
        mesh=_mesh(),
        compiler_params=pltpu.CompilerParams(needs_layout_passes=False),
        scratch_types=[
            pltpu.VMEM((C,), jnp.int32),
            pltpu.VMEM((C, DE), jnp.float32),
            pltpu.VMEM_SHARED((N_PAD, DE), jnp.float32),
        ],
    )
    def k(s_hbm, dst_hbm, out_hbm, id_v, rows_v, acc_sh):
        cid = lax.axis_index("c")
        sid = lax.axis_index("s")

        def zrow(r, _):
            for c in range(DE // L):
                rows_v[r, pl.ds(c * L, L)] = jnp.zeros((L,), jnp.float32)
            return 0

        lax.fori_loop(0, ZC, zrow, 0)
        for p in range(SROWS // ZC):
            pltpu.sync_copy(rows_v.at[pl.ds(0, ZC)],
                            acc_sh.at[pl.ds(sid * SROWS + p * ZC, ZC)])
        plsc.subcore_barrier()

        def chunk(kk, _):
            base = sid * ETILE + kk * C
            pltpu.sync_copy(dst_hbm.at[pl.ds(base, C)], id_v)
            pltpu.sync_copy(s_hbm.at[cid, pl.ds(base, C)], rows_v)
            pltpu.sync_copy(rows_v, acc_sh.at[id_v], add=True)
            return 0

        lax.fori_loop(0, K, chunk, 0)
        plsc.subcore_barrier()
        for p in range(SROWS // ZC):
            r0 = sid * SROWS + p * ZC
            pltpu.sync_copy(acc_sh.at[pl.ds(r0, ZC)],
                            out_hbm.at[cid, pl.ds(r0, ZC)])

    return k(S_stk, dst)


def _sc_agg_scatter(edge_attr_new, col):
    C = 200
    ECORE = E // NC
    ETILE = ECORE // NS
    K = ETILE // C
    ZC = 128
    SROWS = N_PAD // NS

    @functools.partial(
        pl.kernel,
        out_type=jax.ShapeDtypeStruct((NC, N_PAD, DE), jnp.float32),
        mesh=_mesh(),
        compiler_params=pltpu.CompilerParams(needs_layout_passes=False),
        scratch_types=[
            pltpu.VMEM((C,), jnp.int32),
            pltpu.VMEM((C, DE), jnp.float32),
            pltpu.VMEM_SHARED((N_PAD, DE), jnp.float32),
        ],
    )
    def k(ea_hbm, col_hbm, out_hbm, ic_v, rows_v, acc_sh):
        cid = lax.axis_index("c")
        sid = lax.axis_index("s")

        def zrow(r, _):
            for c in range(DE // L):
                rows_v[r, pl.ds(c * L, L)] = jnp.zeros((L,), jnp.float32)
            return 0

        lax.fori_loop(0, ZC, zrow, 0)
        for p in range(SROWS // ZC):
            pltpu.sync_copy(rows_v.at[pl.ds(0, ZC)],
                            acc_sh.at[pl.ds(sid * SROWS + p * ZC, ZC)])
        plsc.subcore_barrier()

        def chunk(kk, _):
            base = cid * ECORE + sid * ETILE + kk * C
            pltpu.sync_copy(col_hbm.at[pl.ds(base, C)], ic_v)
            pltpu.sync_copy(ea_hbm.at[pl.ds(base, C)], rows_v)
            pltpu.sync_copy(rows_v, acc_sh.at[ic_v], add=True)
            return 0

        lax.fori_loop(0, K, chunk, 0)
        plsc.subcore_barrier()
        for p in range(SROWS // ZC):
            r0 = sid * SROWS + p * ZC
            pltpu.sync_copy(acc_sh.at[pl.ds(r0, ZC)],
                            out_hbm.at[cid, pl.ds(r0, ZC)])

    return k(edge_attr_new, col)


def _tc_node_body(x_ref, alo_ref, ahi_ref, p0_ref, p1_ref, den_ref, b_ref,
                  ax_ref, aal_ref, aah_ref, aagg_ref, au_ref, u_ref, bn1_ref,
                  wn2_ref, bn2_ref, wg1u_ref, wg1m_ref, bg1_ref,
                  wg2_ref, bg2_ref, xnew_ref, unew_ref, sum_ref, cnt_ref):
    i = pl.program_id(0)
    x = x_ref[...]
    rden = 1.0 / den_ref[0, 0, :]
    alo = alo_ref[0] * rden[:, None]
    ahi = ahi_ref[0] * rden[:, None]
    p0 = p0_ref[0]
    p1 = p1_ref[0]
    b = b_ref[0, 0, :]
    mask = (b[:, None] == lax.broadcasted_iota(jnp.int32, (x.shape[0], G), 1)
            ).astype(jnp.float32)
    ua8 = jnp.dot(u_ref[...], au_ref[...], preferred_element_type=jnp.float32)
    pre = (jnp.dot(x, ax_ref[...], preferred_element_type=jnp.float32)
           + jnp.dot(alo, aal_ref[...],
                     preferred_element_type=jnp.float32)
           + jnp.dot(ahi, aah_ref[...],
                     preferred_element_type=jnp.float32)
           + jnp.dot(p0 + p1, aagg_ref[...],
                     preferred_element_type=jnp.float32)
           + jnp.dot(mask, ua8) + bn1_ref[...])
    n_h = jax.nn.relu(pre)
    x_new = (jnp.dot(n_h, wn2_ref[...], preferred_element_type=jnp.float32)
             + bn2_ref[...])
    xnew_ref[...] = x_new

    @pl.when(i == 0)
    def _():
        sum_ref[...] = jnp.zeros_like(sum_ref)
        cnt_ref[...] = jnp.zeros_like(cnt_ref)

    sum_ref[...] += lax.dot_general(
        mask, x_new, (((0,), (0,)), ((), ())),
        preferred_element_type=jnp.float32)
    cnt_ref[...] += jnp.broadcast_to(
        jnp.sum(mask, axis=0)[:, None], cnt_ref.shape)

    @pl.when(i == pl.num_programs(0) - 1)
    def _():
        counts = jnp.clip(cnt_ref[:, :1], 1.0, None)
        mean_x = sum_ref[...] / counts
        g_h = jax.nn.relu(
            jnp.dot(u_ref[...], wg1u_ref[...],
                    preferred_element_type=jnp.float32)
            + jnp.dot(mean_x, wg1m_ref[...],
                      preferred_element_type=jnp.float32)
            + bg1_ref[...])
        unew_ref[...] = (
            jnp.dot(g_h, wg2_ref[...], preferred_element_type=jnp.float32)
            + bg2_ref[...])


def _tc_node_global(x, a_halves, agg_part, den3, batch3, A_x, A_a_lo, A_a_hi,
                    A_agg, A_u, u, bn1, Wn2, bn2, Wg1u, Wg1m, bg1, Wg2, bg2):
    nb = 10
    blk = N // nb
    full = lambda shape: pl.BlockSpec(shape, lambda i: tuple(0 for _ in shape))
    return pl.pallas_call(
        _tc_node_body,
        grid=(nb,),
        in_specs=[
            pl.BlockSpec((blk, D), lambda i: (i, 0)),
            pl.BlockSpec((1, blk, DE), lambda i: (0, i, 0)),
            pl.BlockSpec((1, blk, DE), lambda i: (1, i, 0)),
            pl.BlockSpec((1, blk, DE), lambda i: (0, i, 0)),
            pl.BlockSpec((1, blk, DE), lambda i: (1, i, 0)),
            pl.BlockSpec((1, 1, blk), lambda i: (i, 0, 0)),
            pl.BlockSpec((1, 1, blk), lambda i: (i, 0, 0)),
            full((D, H)), full((DE, H)), full((DE, H)), full((DE, H)),
            full((DU, H)), full((G, DU)), full((1, H)),
            full((H, D)), full((1, D)),
            full((DU, H)), full((D, H)), full((1, H)),
            full((H, DU)), full((1, DU)),
        ],
        out_specs=[
            pl.BlockSpec((blk, D), lambda i: (i, 0)),
            pl.BlockSpec((G, DU), lambda i: (0, 0)),
        ],
        out_shape=[
            jax.ShapeDtypeStruct((N, D), jnp.float32),
            jax.ShapeDtypeStruct((G, DU), jnp.float32),
        ],
        scratch_shapes=[
            pltpu.VMEM((G, D), jnp.float32),
            pltpu.VMEM((G, DE), jnp.float32),
        ],
    )(x, a_halves, a_halves, agg_part, agg_part, den3, batch3, A_x, A_a_lo,
      A_a_hi, A_agg, A_u, u, bn1, Wn2, bn2, Wg1u, Wg1m, bg1, Wg2, bg2)


def kernel(x, x_src, edge_index, cg_edge_index, edge_attr, u, batch,
           batch_src, We1, be1, We2, be2, Wn1, bn1, Wn2, bn2,
           Wg1, bg1, Wg2, bg2):
    row = edge_index[0]
    col = edge_index[1]
    src = cg_edge_index[0]
    dst = cg_edge_index[1]
    batch3 = batch.reshape(10, 1, N // 10)

    W_rc = jnp.concatenate([We1[:D], We1[D:2 * D]], axis=1)
    W_attr = We1[2 * D:2 * D + 16]
    W_u = We1[2 * D + 16:]
    A_x = Wn1[:D]
    A_a_lo = Wn1[D:D + DE]
    A_a_hi = Wn1[D + DE:2 * D]
    A_agg = Wn1[2 * D:2 * D + DE]
    A_u = Wn1[2 * D + DE:]
    Wg1u = Wg1[:DU]
    Wg1m = Wg1[DU:]

    P_row, P_col, xn = _tc_pre(
        x, batch3, W_rc, W_u, u, be1.reshape(1, H))

    G_e = _sc_edge_combine(P_row, P_col, row, col)
    edge_attr_new = _tc_edge_mlp(
        G_e, edge_attr, W_attr, We2, be2.reshape(1, DE))

    XD, XS = _sc_gather_pairs(xn, x_src, dst, src)
    ex3, S_stk = _tc_cos_ex(XD, XS)
    den_part = _sc_den_scatter(ex3.reshape(E), dst)
    den = _tc_denom_merge(den_part).reshape(N_PAD)
    a_halves = _sc_a_scatter(S_stk, dst)
    den3 = den[:N].reshape(10, 1, N // 10)

    agg_part = _sc_agg_scatter(edge_attr_new, col)

    x_new, u_new = _tc_node_global(
        x, a_halves, agg_part, den3, batch3,
        A_x, A_a_lo, A_a_hi, A_agg, A_u, u, bn1.reshape(1, H), Wn2,
        bn2.reshape(1, D), Wg1u, Wg1m, bg1.reshape(1, H), Wg2,
        bg2.reshape(1, DU))

    return (x_new, edge_attr_new, u_new)
